# baseline (speedup 1.0000x reference)
import jax
import jax.numpy as jnp
from jax.experimental import pallas as pl
from jax.experimental.pallas import tpu as pltpu

_N_TOKENS = 32768
_DIM = 768
_N_CHUNKS = 8
_CHUNK = _N_TOKENS // _N_CHUNKS


def _dma_kernel(x_hbm, o_hbm, sems):
    for i in range(_N_CHUNKS):
        pltpu.make_async_copy(
            x_hbm.at[pl.ds(i * _CHUNK, _CHUNK), :],
            o_hbm.at[pl.ds(i * _CHUNK, _CHUNK), :],
            sems.at[i],
        ).start()
    for i in range(_N_CHUNKS):
        pltpu.make_async_copy(
            x_hbm.at[pl.ds(i * _CHUNK, _CHUNK), :],
            o_hbm.at[pl.ds(i * _CHUNK, _CHUNK), :],
            sems.at[i],
        ).wait()


def kernel(x, W, b):
    del W, b
    return pl.pallas_call(
        _dma_kernel,
        in_specs=[pl.BlockSpec(memory_space=pl.ANY)],
        out_specs=pl.BlockSpec(memory_space=pl.ANY),
        out_shape=jax.ShapeDtypeStruct((_N_TOKENS, _DIM), jnp.float32),
        scratch_shapes=[pltpu.SemaphoreType.DMA((_N_CHUNKS,))],
    )(x)


# speedup vs baseline: 49.0263x; 49.0263x over previous
import jax
import jax.numpy as jnp
from jax.experimental import pallas as pl

_N_TOKENS = 32768
_DIM = 768
_BLOCK_ROWS = 4096


def _copy_kernel(x_ref, o_ref):
    o_ref[...] = x_ref[...]


def kernel(x, W, b):
    del W, b
    grid = (_N_TOKENS // _BLOCK_ROWS,)
    return pl.pallas_call(
        _copy_kernel,
        grid=grid,
        in_specs=[pl.BlockSpec((_BLOCK_ROWS, _DIM), lambda i: (i, 0))],
        out_specs=pl.BlockSpec((_BLOCK_ROWS, _DIM), lambda i: (i, 0)),
        out_shape=jax.ShapeDtypeStruct((_N_TOKENS, _DIM), jnp.float32),
    )(x)


# speedup vs baseline: 49.2867x; 1.0053x over previous
import jax
import jax.numpy as jnp
from jax.experimental import pallas as pl
from jax.experimental.pallas import tpu as pltpu

_N_TOKENS = 32768
_DIM = 768
_BLOCK_ROWS = 4096


def _copy_kernel(x_ref, o_ref):
    o_ref[...] = x_ref[...]


def kernel(x, W, b):
    del W, b
    grid = (_N_TOKENS // _BLOCK_ROWS,)
    return pl.pallas_call(
        _copy_kernel,
        grid=grid,
        in_specs=[pl.BlockSpec((_BLOCK_ROWS, _DIM), lambda i: (i, 0))],
        out_specs=pl.BlockSpec((_BLOCK_ROWS, _DIM), lambda i: (i, 0)),
        out_shape=jax.ShapeDtypeStruct((_N_TOKENS, _DIM), jnp.float32),
    )(x)
